# baseline (speedup 1.0000x reference)
import functools

import jax
import jax.numpy as jnp
from jax import lax
from jax.experimental import pallas as pl
from jax.experimental.pallas import tpu as pltpu
from jax.experimental.pallas import tpu_sc as plsc

import numpy as np

B = 50000
D = 128
DW = D // 2
L = 16
S = 10
NW = 32
NS = 16
BLK = 112
K0 = 14
K1 = 14
NBLOCKS = NS * (K0 + K1)
B_PAD = NBLOCKS * BLK
NIDX = K0 * (S + 1)
MB = 1024


@functools.cache
def _make_sc_gather():
    @functools.partial(
        pl.kernel,
        out_type=[
            jax.ShapeDtypeStruct((B_PAD // 2, D), jnp.float32),
            jax.ShapeDtypeStruct((B_PAD // 2, D), jnp.float32),
        ],
        mesh=plsc.VectorSubcoreMesh(core_axis_name="c", subcore_axis_name="s"),
        compiler_params=pltpu.CompilerParams(needs_layout_passes=False),
        scratch_types=[
            pltpu.VMEM((NIDX, BLK), jnp.int32),
            pltpu.VMEM((2, BLK, D), jnp.float32),
            pltpu.VMEM((2, BLK, D), jnp.float32),
            pltpu.SemaphoreType.DMA,
            pltpu.SemaphoreType.DMA,
            pltpu.SemaphoreType.DMA,
            pltpu.SemaphoreType.DMA,
            pltpu.SemaphoreType.DMA,
            pltpu.SemaphoreType.DMA,
            pltpu.SemaphoreType.DMA,
            pltpu.SemaphoreType.DMA,
        ],
    )
    def _sc_gather(idx_hbm, feat_hbm, self_out, sum_out,
                   idx_v, selfb, accb, sgs0, sgs1, sga0, sga1,
                   sos0, sos1, soa0, soa1):
        sg_self = (sgs0, sgs1)
        sg_acc = (sga0, sga1)
        so_self = (sos0, sos1)
        so_acc = (soa0, soa1)
        cidx = lax.axis_index("c")
        sidx = lax.axis_index("s")
        wid = sidx * 2 + cidx
        wofs = jnp.where(cidx == 0, sidx * K0, NS * K0 + sidx * K1) * BLK

        pltpu.sync_copy(idx_hbm.at[wid], idx_v)

        zeros = jnp.zeros((L,), jnp.float32)

        def zero_acc(p):
            def zrow(r, c2):
                for c in range(D // L):
                    accb[p, r, pl.ds(c * L, L)] = zeros
                return c2

            lax.fori_loop(0, BLK, zrow, 0)

        def fire_block(b):
            p = b & 1
            zero_acc(p)
            ds = pltpu.async_copy(
                feat_hbm.at[idx_v.at[b * (S + 1)]], selfb.at[p], sg_self[p])
            da = [
                pltpu.async_copy(
                    feat_hbm.at[idx_v.at[b * (S + 1) + 1 + j]], accb.at[p],
                    sg_acc[p], add=True)
                for j in range(S)
            ]
            return ds, da

        def pack_buf(src, p):
            def prow(rp, c2):
                for half in range(2):
                    r = rp * 2 + half
                    for c in range(D // 32):
                        a = src[p, r, pl.ds(c * 32, L)]
                        bv = src[p, r, pl.ds(c * 32 + L, L)]
                        pk = plsc.pack(
                            a, bv, format=plsc.PackFormat.INTERLEAVED)
                        src[p, rp, pl.ds(half * DW + c * L, L)] = (
                            plsc.bitcast(pk, jnp.float32))
                return c2

            lax.fori_loop(0, BLK // 2, prow, 0)

        def retire_block(b, gathers):
            p = b & 1
            base2 = pl.multiple_of((wofs + b * BLK) // 2, 8)
            ds, da = gathers
            ds.wait()
            pack_buf(selfb, p)
            os = pltpu.async_copy(
                selfb.at[p, pl.ds(0, BLK // 2)],
                self_out.at[pl.ds(base2, BLK // 2)], so_self[p])
            for d in da:
                d.wait()
            pack_buf(accb, p)
            oa = pltpu.async_copy(
                accb.at[p, pl.ds(0, BLK // 2)],
                sum_out.at[pl.ds(base2, BLK // 2)], so_acc[p])
            return os, oa

        def pipeline(first, nblk):
            outs = [None, None]
            gathers = fire_block(first)
            for i in range(nblk):
                b = first + i
                nxt = None
                if i + 1 < nblk:
                    p = (b + 1) & 1
                    if outs[p] is not None:
                        outs[p][0].wait()
                        outs[p][1].wait()
                    nxt = fire_block(b + 1)
                outs[b & 1] = retire_block(b, gathers)
                gathers = nxt
            for dd in outs:
                dd[0].wait()
                dd[1].wait()

        pipeline(0, K1)
        if K0 > K1:
            @pl.when(cidx == 0)
            def _extra():
                pipeline(K1, K0 - K1)

    return _sc_gather


_MASK = np.int32(-65536)


def _mm_body(self_ref, sum_ref, w_ref, o_ref):
    si = lax.bitcast_convert_type(self_ref[...], jnp.int32)
    su = lax.bitcast_convert_type(sum_ref[...], jnp.int32)
    sa = lax.bitcast_convert_type(si << 16, jnp.float32)
    sb = lax.bitcast_convert_type(si & _MASK, jnp.float32)
    inv_s = jnp.float32(1.0 / S)
    ua = lax.bitcast_convert_type(su << 16, jnp.float32) * inv_s
    ub = lax.bitcast_convert_type(su & _MASK, jnp.float32) * inv_s
    acc = jnp.dot(sa, w_ref[:DW, :], preferred_element_type=jnp.float32)
    acc += jnp.dot(sb, w_ref[DW:D, :], preferred_element_type=jnp.float32)
    acc += jnp.dot(ua, w_ref[D:D + DW, :], preferred_element_type=jnp.float32)
    acc += jnp.dot(ub, w_ref[D + DW:, :], preferred_element_type=jnp.float32)
    o_ref[...] = jnp.maximum(acc, 0.0)


_PA = np.array([32 * (k // 16) + (k % 16) for k in range(DW)])
_PB = _PA + 16


def _build_idx(nodes, neigh_idx):
    pad = B_PAD - B
    nodes_b = jnp.pad(nodes, (0, pad)).reshape(NBLOCKS, 1, BLK)
    neigh_b = (jnp.pad(neigh_idx, ((0, pad), (0, 0)))
               .reshape(NBLOCKS, BLK, S)
               .transpose(0, 2, 1))
    blocks = jnp.concatenate([nodes_b, neigh_b], axis=1)
    per_worker = []
    for wid in range(NW):
        s, c = wid // 2, wid % 2
        if c == 0:
            w = blocks[s * K0:(s + 1) * K0].reshape(NIDX, BLK)
        else:
            w = blocks[NS * K0 + s * K1:NS * K0 + (s + 1) * K1]
            w = jnp.pad(w.reshape(K1 * (S + 1), BLK),
                        ((0, (K0 - K1) * (S + 1)), (0, 0)))
        per_worker.append(w)
    return jnp.stack(per_worker)


def kernel(nodes, neigh_idx, features, weight):
    idx_all = _build_idx(nodes, neigh_idx)
    self_h, sum_h = _make_sc_gather()(idx_all, features)
    self_pk = self_h.reshape(B_PAD, DW)
    sum_pk = sum_h.reshape(B_PAD, DW)
    w1 = weight[:D]
    w2 = weight[D:]
    w_perm = jnp.concatenate([w1[_PA], w1[_PB], w2[_PA], w2[_PB]], axis=0)
    out = pl.pallas_call(
        _mm_body,
        grid=((B + MB - 1) // MB,),
        in_specs=[
            pl.BlockSpec((MB, DW), lambda i: (i, 0)),
            pl.BlockSpec((MB, DW), lambda i: (i, 0)),
            pl.BlockSpec((2 * D, D), lambda i: (0, 0)),
        ],
        out_specs=pl.BlockSpec((MB, D), lambda i: (i, 0)),
        out_shape=jax.ShapeDtypeStruct((B, D), jnp.float32),
    )(self_pk, sum_pk, w_perm)
    return out


# speedup vs baseline: 1.1446x; 1.1446x over previous
import functools

import jax
import jax.numpy as jnp
from jax import lax
from jax.experimental import pallas as pl
from jax.experimental.pallas import tpu as pltpu
from jax.experimental.pallas import tpu_sc as plsc

B = 50000
D = 128
L = 16
S = 10
NW = 32
NS = 16
BLK = 112
K0 = 14
K1 = 14
NBLOCKS = NS * (K0 + K1)
B_PAD = NBLOCKS * BLK
NIDX = K0 * (S + 1)
MB = 1024


@functools.cache
def _make_sc_gather():
    @functools.partial(
        pl.kernel,
        out_type=[
            jax.ShapeDtypeStruct((B_PAD, D), jnp.float32),
            jax.ShapeDtypeStruct((B_PAD, D), jnp.float32),
        ],
        mesh=plsc.VectorSubcoreMesh(core_axis_name="c", subcore_axis_name="s"),
        scratch_types=[
            pltpu.VMEM((NIDX, BLK), jnp.int32),
            pltpu.VMEM((2, BLK, D), jnp.float32),
            pltpu.VMEM((2, BLK, D), jnp.float32),
            pltpu.SemaphoreType.DMA,
            pltpu.SemaphoreType.DMA,
            pltpu.SemaphoreType.DMA,
            pltpu.SemaphoreType.DMA,
            pltpu.SemaphoreType.DMA,
            pltpu.SemaphoreType.DMA,
            pltpu.SemaphoreType.DMA,
            pltpu.SemaphoreType.DMA,
        ],
    )
    def _sc_gather(idx_hbm, feat_hbm, self_out, sum_out,
                   idx_v, selfb, accb, sgs0, sgs1, sga0, sga1,
                   sos0, sos1, soa0, soa1):
        sg_self = (sgs0, sgs1)
        sg_acc = (sga0, sga1)
        so_self = (sos0, sos1)
        so_acc = (soa0, soa1)
        cidx = lax.axis_index("c")
        sidx = lax.axis_index("s")
        wid = sidx * 2 + cidx
        wofs = jnp.where(cidx == 0, sidx * K0, NS * K0 + sidx * K1) * BLK

        pltpu.sync_copy(idx_hbm.at[wid], idx_v)

        zeros = jnp.zeros((L,), jnp.float32)

        def zero_acc(p):
            def zrow(r, c2):
                for c in range(D // L):
                    accb[p, r, pl.ds(c * L, L)] = zeros
                return c2

            lax.fori_loop(0, BLK, zrow, 0)

        def fire_block(b):
            p = b & 1
            zero_acc(p)
            ds = pltpu.async_copy(
                feat_hbm.at[idx_v.at[b * (S + 1)]], selfb.at[p], sg_self[p])
            da = [
                pltpu.async_copy(
                    feat_hbm.at[idx_v.at[b * (S + 1) + 1 + j]], accb.at[p],
                    sg_acc[p], add=True)
                for j in range(S)
            ]
            return ds, da

        def retire_block(b, gathers):
            p = b & 1
            base = wofs + b * BLK
            ds, da = gathers
            ds.wait()
            os = pltpu.async_copy(selfb.at[p], self_out.at[pl.ds(base, BLK)],
                                  so_self[p])
            for d in da:
                d.wait()
            oa = pltpu.async_copy(accb.at[p], sum_out.at[pl.ds(base, BLK)],
                                  so_acc[p])
            return os, oa

        def pipeline(first, nblk):
            outs = [None, None]
            gathers = fire_block(first)
            for i in range(nblk):
                b = first + i
                nxt = None
                if i + 1 < nblk:
                    p = (b + 1) & 1
                    if outs[p] is not None:
                        outs[p][0].wait()
                        outs[p][1].wait()
                    nxt = fire_block(b + 1)
                outs[b & 1] = retire_block(b, gathers)
                gathers = nxt
            for dd in outs:
                dd[0].wait()
                dd[1].wait()

        pipeline(0, K1)
        if K0 > K1:
            @pl.when(cidx == 0)
            def _extra():
                pipeline(K1, K0 - K1)

    return _sc_gather


def _mm_body(self_ref, sum_ref, w_ref, o_ref):
    w1 = w_ref[:D, :]
    w2 = w_ref[D:, :]
    x2 = sum_ref[...] * jnp.float32(1.0 / S)
    acc = jnp.dot(self_ref[...], w1, preferred_element_type=jnp.float32)
    acc += jnp.dot(x2, w2, preferred_element_type=jnp.float32)
    o_ref[...] = jnp.maximum(acc, 0.0)


def _build_idx(nodes, neigh_idx):
    pad = B_PAD - B
    nodes_b = jnp.pad(nodes, (0, pad)).reshape(NBLOCKS, 1, BLK)
    neigh_b = (jnp.pad(neigh_idx, ((0, pad), (0, 0)))
               .reshape(NBLOCKS, BLK, S)
               .transpose(0, 2, 1))
    blocks = jnp.concatenate([nodes_b, neigh_b], axis=1)
    per_worker = []
    for wid in range(NW):
        s, c = wid // 2, wid % 2
        if c == 0:
            w = blocks[s * K0:(s + 1) * K0].reshape(NIDX, BLK)
        else:
            w = blocks[NS * K0 + s * K1:NS * K0 + (s + 1) * K1]
            w = jnp.pad(w.reshape(K1 * (S + 1), BLK),
                        ((0, (K0 - K1) * (S + 1)), (0, 0)))
        per_worker.append(w)
    return jnp.stack(per_worker)


def kernel(nodes, neigh_idx, features, weight):
    idx_all = _build_idx(nodes, neigh_idx)
    self_feats, neigh_sum = _make_sc_gather()(idx_all, features)
    out = pl.pallas_call(
        _mm_body,
        grid=((B + MB - 1) // MB,),
        in_specs=[
            pl.BlockSpec((MB, D), lambda i: (i, 0)),
            pl.BlockSpec((MB, D), lambda i: (i, 0)),
            pl.BlockSpec((2 * D, D), lambda i: (0, 0)),
        ],
        out_specs=pl.BlockSpec((MB, D), lambda i: (i, 0)),
        out_shape=jax.ShapeDtypeStruct((B, D), jnp.float32),
    )(self_feats, neigh_sum, weight)
    return out
